# baseline (speedup 1.0000x reference)
import functools

import jax
import jax.numpy as jnp
from jax import lax
from jax.experimental import pallas as pl
from jax.experimental.pallas import tpu as pltpu
from jax.experimental.pallas import tpu_sc as plsc

B = 16384
A = 200
NC = 2
NS = 16
L = 16
NW = NC * NS
RPW = B // NW
GROUPS = RPW // L


def _take16(table_vec, idx):
    return lax.gather(
        table_vec,
        idx[:, None],
        dimension_numbers=lax.GatherDimensionNumbers(
            offset_dims=(),
            collapsed_slice_dims=(0,),
            start_index_map=(0,),
        ),
        slice_sizes=(1,),
        mode=lax.GatherScatterMode.PROMISE_IN_BOUNDS,
    )


def _sae_body(energies_hbm, species_hbm, table_hbm, out_hbm,
              species_v, energies_v, out_v, table_v):
    wid = lax.axis_index("s") * NC + lax.axis_index("c")
    base = wid * RPW

    pltpu.sync_copy(table_hbm, table_v)
    pltpu.sync_copy(energies_hbm.at[pl.ds(base, RPW)], energies_v)
    pltpu.sync_copy(species_hbm.at[pl.ds(base * A, RPW * A)], species_v)

    table_vec = table_v[...]
    iota_rows = lax.iota(jnp.int32, L) * A
    zeros_f = jnp.zeros((L,), jnp.float32)

    def group_fn(g, _):
        idx0 = iota_rows + g * (L * A)

        def atom_fn(a, carry):
            idx, acc = carry
            s = plsc.load_gather(species_v, [idx])
            vals = _take16(table_vec, s)
            return idx + 1, acc + vals

        _, acc = lax.fori_loop(0, A, atom_fn, (idx0, zeros_f), unroll=8)
        e = energies_v[pl.ds(g * L, L)]
        out_v[pl.ds(g * L, L)] = e - acc
        return 0

    lax.fori_loop(0, GROUPS, group_fn, 0)
    pltpu.sync_copy(out_v, out_hbm.at[pl.ds(base, RPW)])


@jax.jit
def _sae_kernel(energies, species_flat, table16):
    mesh = plsc.VectorSubcoreMesh(
        core_axis_name="c", subcore_axis_name="s",
        num_cores=NC, num_subcores=NS,
    )
    f = functools.partial(
        pl.kernel,
        mesh=mesh,
        compiler_params=pltpu.CompilerParams(needs_layout_passes=False),
        out_type=jax.ShapeDtypeStruct((B,), jnp.float32),
        scratch_types=[
            pltpu.VMEM((RPW * A,), jnp.int32),
            pltpu.VMEM((RPW,), jnp.float32),
            pltpu.VMEM((RPW,), jnp.float32),
            pltpu.VMEM((L,), jnp.float32),
        ],
    )(_sae_body)
    return f(energies, species_flat, table16)


def kernel(energies, species, self_energies):
    species_flat = species.reshape(-1).astype(jnp.int32)
    table16 = jnp.zeros((L,), jnp.float32).at[: self_energies.shape[0]].set(
        self_energies.astype(jnp.float32))
    return _sae_kernel(energies, species_flat, table16)


# speedup vs baseline: 1.0809x; 1.0809x over previous
import functools

import jax
import jax.numpy as jnp
from jax import lax
from jax.experimental import pallas as pl
from jax.experimental.pallas import tpu as pltpu
from jax.experimental.pallas import tpu_sc as plsc

B = 16384
A = 200
NC = 2
NS = 16
L = 16
NW = NC * NS
RPW = B // NW
GROUPS = RPW // L


def _take16(table_vec, idx):
    return lax.gather(
        table_vec,
        idx[:, None],
        dimension_numbers=lax.GatherDimensionNumbers(
            offset_dims=(),
            collapsed_slice_dims=(0,),
            start_index_map=(0,),
        ),
        slice_sizes=(1,),
        mode=lax.GatherScatterMode.PROMISE_IN_BOUNDS,
    )


def _sae_body(energies_hbm, species_hbm, table_hbm, out_hbm,
              species_v, energies_v, out_v, table_v):
    wid = lax.axis_index("s") * NC + lax.axis_index("c")
    base = wid * RPW

    pltpu.sync_copy(table_hbm, table_v)
    pltpu.sync_copy(energies_hbm.at[pl.ds(base, RPW)], energies_v)
    pltpu.sync_copy(species_hbm.at[pl.ds(base * A, RPW * A)], species_v)

    table_vec = table_v[...]
    iota = lax.iota(jnp.int32, L)
    stagger = lax.shift_right_logical(iota, 1)
    idx_start = iota * A + stagger
    zeros_f = jnp.zeros((L,), jnp.float32)
    U = 8
    MAIN = (A - U) // U
    tail_off = [
        jnp.where(stagger + k >= U, (A - U) + k - A, (A - U) + k)
        for k in range(U)
    ]

    def group_fn(g, _):
        idx0 = idx_start + g * (L * A)

        def atom_fn(_, carry):
            idx, accs = carry
            new = []
            for k in range(U):
                s = plsc.load_gather(species_v, [idx + k])
                new.append(accs[k] + _take16(table_vec, s))
            return idx + U, tuple(new)

        idx, accs = lax.fori_loop(
            0, MAIN, atom_fn, (idx0, (zeros_f,) * U))
        accs = list(accs)
        for k in range(U):
            s = plsc.load_gather(species_v, [idx0 + tail_off[k]])
            accs[k] = accs[k] + _take16(table_vec, s)
        acc = ((accs[0] + accs[1]) + (accs[2] + accs[3])) + (
            (accs[4] + accs[5]) + (accs[6] + accs[7]))
        e = energies_v[pl.ds(g * L, L)]
        out_v[pl.ds(g * L, L)] = e - acc
        return 0

    lax.fori_loop(0, GROUPS, group_fn, 0)
    pltpu.sync_copy(out_v, out_hbm.at[pl.ds(base, RPW)])


@jax.jit
def _sae_kernel(energies, species_flat, table16):
    mesh = plsc.VectorSubcoreMesh(
        core_axis_name="c", subcore_axis_name="s",
        num_cores=NC, num_subcores=NS,
    )
    f = functools.partial(
        pl.kernel,
        mesh=mesh,
        compiler_params=pltpu.CompilerParams(needs_layout_passes=False),
        out_type=jax.ShapeDtypeStruct((B,), jnp.float32),
        scratch_types=[
            pltpu.VMEM((RPW * A,), jnp.int32),
            pltpu.VMEM((RPW,), jnp.float32),
            pltpu.VMEM((RPW,), jnp.float32),
            pltpu.VMEM((L,), jnp.float32),
        ],
    )(_sae_body)
    return f(energies, species_flat, table16)


def kernel(energies, species, self_energies):
    species_flat = species.reshape(-1).astype(jnp.int32)
    table16 = jnp.zeros((L,), jnp.float32).at[: self_energies.shape[0]].set(
        self_energies.astype(jnp.float32))
    return _sae_kernel(energies, species_flat, table16)


# speedup vs baseline: 1.6388x; 1.5161x over previous
import functools

import jax
import jax.numpy as jnp
from jax import lax
from jax.experimental import pallas as pl
from jax.experimental.pallas import tpu as pltpu
from jax.experimental.pallas import tpu_sc as plsc

B = 16384
A = 200
NC = 2
NS = 16
L = 16
NW = NC * NS
RPW = B // NW
CHUNK = 128
NCH = RPW // CHUNK
CGROUPS = CHUNK // L
U = 8
MAIN = (A - 2 * U) // U
TAIL = A - MAIN * U


def _take16(table_vec, idx):
    return lax.gather(
        table_vec,
        idx[:, None],
        dimension_numbers=lax.GatherDimensionNumbers(
            offset_dims=(),
            collapsed_slice_dims=(0,),
            start_index_map=(0,),
        ),
        slice_sizes=(1,),
        mode=lax.GatherScatterMode.PROMISE_IN_BOUNDS,
    )


def _sae_body(energies_hbm, species_hbm, table_hbm, out_hbm,
              buf0, buf1, energies_v, out_v, table_v, sem0, sem1):
    wid = lax.axis_index("s") * NC + lax.axis_index("c")
    base = wid * RPW

    bufs = (buf0, buf1)
    sems = (sem0, sem1)

    def chunk_copy(c, nbuf):
        return pltpu.make_async_copy(
            species_hbm.at[pl.ds(base + c * CHUNK, CHUNK)],
            bufs[nbuf], sems[nbuf])

    chunk_copy(0, 0).start()
    pltpu.sync_copy(table_hbm, table_v)
    pltpu.sync_copy(energies_hbm.at[pl.ds(base, RPW)], energies_v)

    table_vec = table_v[...]
    iota = lax.iota(jnp.int32, L)
    zeros_f = jnp.zeros((L,), jnp.float32)
    tail_col = [
        jnp.where(iota + k >= TAIL, iota + MAIN * U + k - A,
                  iota + MAIN * U + k)
        for k in range(TAIL)
    ]

    def do_chunk(c, nbuf):
        chunk_copy(c, nbuf).wait()
        species_v = bufs[nbuf]

        def group_fn(g, _):
            rows = iota + g * L

            def atom_fn(_, carry):
                col, accs = carry
                new = []
                for k in range(U):
                    s = plsc.load_gather(species_v, [rows, col + k])
                    new.append(accs[k] + _take16(table_vec, s))
                return col + U, tuple(new)

            _, accs = lax.fori_loop(
                0, MAIN, atom_fn, (iota, (zeros_f,) * U))
            accs = list(accs)
            for k in range(TAIL):
                s = plsc.load_gather(species_v, [rows, tail_col[k]])
                accs[k % U] = accs[k % U] + _take16(table_vec, s)
            acc = ((accs[0] + accs[1]) + (accs[2] + accs[3])) + (
                (accs[4] + accs[5]) + (accs[6] + accs[7]))
            off = c * CHUNK + g * L
            e = energies_v[pl.ds(off, L)]
            out_v[pl.ds(off, L)] = e - acc
            return 0

        lax.fori_loop(0, CGROUPS, group_fn, 0)

    for c in range(NCH):
        if c + 1 < NCH:
            chunk_copy(c + 1, (c + 1) % 2).start()
        do_chunk(c, c % 2)

    pltpu.sync_copy(out_v, out_hbm.at[pl.ds(base, RPW)])


@jax.jit
def _sae_kernel(energies, species, table16):
    mesh = plsc.VectorSubcoreMesh(
        core_axis_name="c", subcore_axis_name="s",
        num_cores=NC, num_subcores=NS,
    )
    f = functools.partial(
        pl.kernel,
        mesh=mesh,
        compiler_params=pltpu.CompilerParams(needs_layout_passes=False),
        out_type=jax.ShapeDtypeStruct((B,), jnp.float32),
        scratch_types=[
            pltpu.VMEM((CHUNK, A), jnp.int32),
            pltpu.VMEM((CHUNK, A), jnp.int32),
            pltpu.VMEM((RPW,), jnp.float32),
            pltpu.VMEM((RPW,), jnp.float32),
            pltpu.VMEM((L,), jnp.float32),
            pltpu.SemaphoreType.DMA,
            pltpu.SemaphoreType.DMA,
        ],
    )(_sae_body)
    return f(energies, species, table16)


def kernel(energies, species, self_energies):
    table16 = jnp.zeros((L,), jnp.float32).at[: self_energies.shape[0]].set(
        self_energies.astype(jnp.float32))
    return _sae_kernel(energies, species.astype(jnp.int32), table16)
